# Initial kernel scaffold; baseline (speedup 1.0000x reference)
import jax
import jax.numpy as jnp
from jax.experimental import pallas as pl


def kernel(ts, pos_embeddings):
    raise NotImplementedError("write your pallas kernel here")


# speedup vs baseline: 8.5767x; 8.5767x over previous
import functools

import jax
import jax.numpy as jnp
from jax.experimental import pallas as pl

SEQ = 200
EXPAND = 16
K_CH = 8
N_CH = SEQ // K_CH


def _body(ts_ref, tab_ref, out_ref):
    ts = ts_ref[...]
    b = ts.shape[0]
    sb = jax.lax.bitcast_convert_type(ts, jnp.int32)
    s = jnp.where(sb < 0, sb ^ jnp.int32(0x7FFFFFFF), sb)

    s3 = s[:, None, :]
    jiota = jax.lax.broadcasted_iota(jnp.int32, (1, K_CH, SEQ), 2)
    kiota = jax.lax.broadcasted_iota(jnp.int32, (1, K_CH, SEQ), 1)
    acc = jnp.zeros((b, K_CH, SEQ), jnp.int32)
    for q in range(N_CH):
        sk = jax.lax.slice(s, (0, q * K_CH), (b, (q + 1) * K_CH))
        m = (jiota > kiota + (q * K_CH)).astype(jnp.int32)
        cmp = (sk[:, :, None] - m) < s3
        acc = acc + cmp.astype(jnp.int32)
    rank = jnp.sum(acc, axis=1)

    iota_i = jax.lax.broadcasted_iota(jnp.int32, (b, SEQ, SEQ), 1)
    m3 = (rank[:, None, :] == iota_i).astype(jnp.float32)
    m2 = m3.reshape(b * SEQ, SEQ)
    out = jnp.dot(m2, tab_ref[...], preferred_element_type=jnp.float32)
    out_ref[...] = out.reshape(b, SEQ, EXPAND)


@functools.partial(jax.jit, static_argnames=("block_b",))
def _run(ts, table, block_b=64):
    batch = ts.shape[0]
    grid = (batch // block_b,)
    return pl.pallas_call(
        _body,
        grid=grid,
        in_specs=[
            pl.BlockSpec((block_b, SEQ), lambda i: (i, 0)),
            pl.BlockSpec((SEQ, EXPAND), lambda i: (0, 0)),
        ],
        out_specs=pl.BlockSpec((block_b, SEQ, EXPAND), lambda i: (i, 0, 0)),
        out_shape=jax.ShapeDtypeStruct((batch, SEQ, EXPAND), jnp.float32),
    )(ts, table)


def kernel(ts, pos_embeddings):
    return _run(ts, pos_embeddings)
